# Initial kernel scaffold; baseline (speedup 1.0000x reference)
import jax
import jax.numpy as jnp
from jax.experimental import pallas as pl


def kernel(feature, X, indices, values, weight, unique_idx, W_mask, W_ctx):
    raise NotImplementedError("write your pallas kernel here")


# speedup vs baseline: 7.8540x; 7.8540x over previous
import functools

import jax
import jax.numpy as jnp
from jax import lax
from jax.experimental import pallas as pl
from jax.experimental.pallas import tpu as pltpu
from jax.experimental.pallas import tpu_sc as plsc

N = 10000
L = 16
E = 320000
D = 128
M = 5000

NPAD = N + L
NC = 2
NS = 16
NW = NC * NS
EW = E // NW
CH = 200
NCHUNK = EW // CH
MPAD = 5120
MW = MPAD // NW
APAD = 10112
RPS = APAD // NS


def _tc_prompt_body(w_ref, xp_ref, p_ref):
    w = w_ref[...]
    mx = jnp.max(w, axis=-1, keepdims=True)
    ex = jnp.exp(w - mx)
    sm = ex / jnp.sum(ex, axis=-1, keepdims=True)
    p_ref[...] = jnp.dot(sm, xp_ref[...], preferred_element_type=jnp.float32)


def _tc_out_body(agg2_ref, wctx_ref, feat_ref, wmask_ref, ctx_ref, mask_ref):
    agg = agg2_ref[0, :NPAD] + agg2_ref[1, :NPAD]
    ctx = jnp.dot(agg, wctx_ref[...], preferred_element_type=jnp.float32)
    ctx_ref[...] = jnp.maximum(ctx, 0.0)
    msk = jnp.dot(feat_ref[...], wmask_ref[...], preferred_element_type=jnp.float32)
    mask_ref[...] = jnp.maximum(msk, 0.0)


def _sc_body(rows_hbm, cols_hbm, vals_hbm, x_hbm, p_hbm, uidx_hbm, out_hbm,
             cols_v, rows_v, vals_v, gbuf, puidx_v, acc, sem):
    c = lax.axis_index("c")
    s = lax.axis_index("s")
    w = s * NC + c

    def _zero_row(i, _):
        for j in range(D // 16):
            gbuf[i, pl.ds(j * 16, 16)] = jnp.zeros((16,), jnp.float32)
        return _
    lax.fori_loop(0, CH, _zero_row, None)
    for t in range(RPS // CH):
        pltpu.sync_copy(gbuf, acc.at[pl.ds(s * RPS + t * CH, CH)])
    rem = RPS % CH
    if rem:
        pltpu.sync_copy(gbuf.at[pl.ds(0, rem)],
                        acc.at[pl.ds(s * RPS + (RPS // CH) * CH, rem)])
    plsc.subcore_barrier()

    pltpu.sync_copy(p_hbm.at[pl.ds(w * MW, MW)], gbuf.at[pl.ds(0, MW)])
    pltpu.sync_copy(uidx_hbm.at[pl.ds(w * MW, MW)], puidx_v)
    pltpu.sync_copy(gbuf.at[pl.ds(0, MW)], acc.at[puidx_v], add=True)

    def _chunk(k, _):
        base = w * EW + k * CH
        pltpu.sync_copy(cols_hbm.at[pl.ds(base, CH)], cols_v)
        pltpu.sync_copy(rows_hbm.at[pl.ds(base, CH)], rows_v)
        pltpu.sync_copy(vals_hbm.at[pl.ds(base, CH)], vals_v.at[pl.ds(0, CH)])
        pltpu.async_copy(x_hbm.at[cols_v], gbuf, sem).wait()

        def _scale16(g, nlanes):
            v16 = vals_v[pl.ds(g * 16, 16)]
            for e16 in range(nlanes):
                e = g * 16 + e16
                vb = jnp.full((16,), v16[e16], jnp.float32)
                for j in range(D // 16):
                    sl = (e, pl.ds(j * 16, 16))
                    gbuf[sl] = gbuf[sl] * vb

        def _scale(g, _):
            _scale16(g, 16)
            return _
        lax.fori_loop(0, CH // 16, _scale, None)
        if CH % 16:
            _scale16(CH // 16, CH % 16)

        pltpu.sync_copy(gbuf, acc.at[rows_v], add=True)
        return _
    lax.fori_loop(0, NCHUNK, _chunk, None)
    plsc.subcore_barrier()

    pltpu.sync_copy(acc.at[pl.ds(s * RPS, RPS)], out_hbm.at[c, pl.ds(s * RPS, RPS)])


_sc_agg = functools.partial(
    pl.kernel,
    out_type=jax.ShapeDtypeStruct((NC, APAD, D), jnp.float32),
    mesh=plsc.VectorSubcoreMesh(core_axis_name="c", subcore_axis_name="s"),
    scratch_types=[
        pltpu.VMEM((CH,), jnp.int32),
        pltpu.VMEM((CH,), jnp.int32),
        pltpu.VMEM((CH + 16, ), jnp.float32),
        pltpu.VMEM((CH, D), jnp.float32),
        pltpu.VMEM((MW,), jnp.int32),
        pltpu.VMEM_SHARED((APAD, D), jnp.float32),
        pltpu.SemaphoreType.DMA,
    ],
)(_sc_body)


def kernel(feature, X, indices, values, weight, unique_idx, W_mask, W_ctx):
    rows = indices[0]
    cols = indices[1]
    xp = X[N:]

    prompt = pl.pallas_call(
        _tc_prompt_body,
        out_shape=jax.ShapeDtypeStruct((M, D), jnp.float32),
    )(weight, xp)
    p_pad = jnp.pad(prompt, ((0, MPAD - M), (0, 0)))
    uidx_pad = jnp.concatenate(
        [unique_idx.astype(jnp.int32), jnp.arange(MPAD - M, dtype=jnp.int32)])

    agg2 = _sc_agg(rows, cols, values, X, p_pad, uidx_pad)

    pred_context, pred_mask = pl.pallas_call(
        _tc_out_body,
        out_shape=(
            jax.ShapeDtypeStruct((NPAD, D), jnp.float32),
            jax.ShapeDtypeStruct((N, D), jnp.float32),
        ),
    )(agg2, W_ctx, feature, W_mask)

    return (pred_mask, pred_context, pred_mask[-L:], pred_context[-L:], weight)


# speedup vs baseline: 12.0448x; 1.5336x over previous
import functools

import jax
import jax.numpy as jnp
from jax import lax
from jax.experimental import pallas as pl
from jax.experimental.pallas import tpu as pltpu
from jax.experimental.pallas import tpu_sc as plsc

N = 10000
L = 16
E = 320000
D = 128
M = 5000

NPAD = N + L
NC = 2
NS = 16
NW = NC * NS
CH = 128
EP = 327680
EW = EP // NW
NCHUNK = EW // CH
G = 16
NG = NCHUNK // G
MPAD = 5120
MW = MPAD // NW
APAD = 10112
RPS = APAD // NS


def _tc_prompt_body(w_ref, xp_ref, p_ref):
    w = w_ref[...]
    mx = jnp.max(w, axis=-1, keepdims=True)
    ex = jnp.exp(w - mx)
    sm = ex / jnp.sum(ex, axis=-1, keepdims=True)
    p_ref[...] = jnp.dot(sm, xp_ref[...], preferred_element_type=jnp.float32)


def _tc_out_body(agg2_ref, wctx_ref, feat_ref, wmask_ref, ctx_ref, mask_ref):
    agg = agg2_ref[0, :NPAD] + agg2_ref[1, :NPAD]
    ctx = jnp.dot(agg, wctx_ref[...], preferred_element_type=jnp.float32)
    ctx_ref[...] = jnp.maximum(ctx, 0.0)
    msk = jnp.dot(feat_ref[...], wmask_ref[...], preferred_element_type=jnp.float32)
    mask_ref[...] = jnp.maximum(msk, 0.0)


def _sc_body(rows_hbm, cols_hbm, vals_hbm, x_hbm, p_hbm, uidx_hbm, out_hbm,
             colsg, rowsg, valsg, gbuf, puidx2, acc,
             semg0, semg1, sems0, sems1):
    c = lax.axis_index("c")
    s = lax.axis_index("s")
    w = s * NC + c

    def g_issue(k, b, sem):
        pltpu.async_copy(x_hbm.at[colsg.at[k]], gbuf.at[b], sem)

    def sc_issue(k, b, sem):
        pltpu.async_copy(gbuf.at[b], acc.at[rowsg.at[k]], sem, add=True)

    def dwait(sem):
        pltpu.make_async_copy(x_hbm.at[pl.ds(0, CH)], gbuf.at[0], sem).wait()

    def load_idx_group(gi):
        r0 = w * NCHUNK + gi * G
        pltpu.sync_copy(cols_hbm.at[pl.ds(r0, G)], colsg)
        pltpu.sync_copy(rows_hbm.at[pl.ds(r0, G)], rowsg)
        pltpu.sync_copy(vals_hbm.at[pl.ds(r0, G)], valsg)

    def scale_chunk(b, k):
        def _sg(g, carry):
            v16 = valsg[k, pl.ds(g * 16, 16)]
            for e16 in range(16):
                vb = jnp.full((16,), v16[e16], jnp.float32)
                e = g * 16 + e16
                for j in range(D // 16):
                    sl = (b, e, pl.ds(j * 16, 16))
                    gbuf[sl] = gbuf[sl] * vb
            return carry
        lax.fori_loop(0, CH // 16, _sg, None)

    def _zero_row(i, carry):
        for j in range(D // 16):
            gbuf[0, i, pl.ds(j * 16, 16)] = jnp.zeros((16,), jnp.float32)
        return carry
    lax.fori_loop(0, CH, _zero_row, None)
    for t in range(RPS // CH):
        pltpu.sync_copy(gbuf.at[0], acc.at[pl.ds(s * RPS + t * CH, CH)])
    rem = RPS % CH
    if rem:
        pltpu.sync_copy(gbuf.at[0, pl.ds(0, rem)],
                        acc.at[pl.ds(s * RPS + (RPS // CH) * CH, rem)])
    plsc.subcore_barrier()

    for t in range(2):
        pltpu.sync_copy(p_hbm.at[pl.ds(w * MW + t * 80, 80)],
                        gbuf.at[0, pl.ds(0, 80)])
        pltpu.sync_copy(uidx_hbm.at[pl.ds(w * MW + t * 80, 80)], puidx2.at[t])
        pltpu.sync_copy(gbuf.at[0, pl.ds(0, 80)], acc.at[puidx2.at[t]], add=True)

    def _pair_mid(p, carry):
        a = 2 * p
        dwait(sems1)
        g_issue(a + 1, 1, semg1)
        dwait(semg0)
        scale_chunk(0, a)
        sc_issue(a, 0, sems0)
        dwait(semg1)
        scale_chunk(1, a + 1)
        dwait(sems0)
        g_issue(a + 2, 0, semg0)
        sc_issue(a + 1, 1, sems1)
        return carry

    def _group(gi, carry):
        g_issue(1, 1, semg1)
        dwait(semg0)
        scale_chunk(0, 0)
        sc_issue(0, 0, sems0)
        dwait(semg1)
        scale_chunk(1, 1)
        dwait(sems0)
        g_issue(2, 0, semg0)
        sc_issue(1, 1, sems1)

        lax.fori_loop(1, G // 2 - 1, _pair_mid, None)

        dwait(sems1)
        g_issue(G - 1, 1, semg1)
        dwait(semg0)
        scale_chunk(0, G - 2)
        sc_issue(G - 2, 0, sems0)
        dwait(semg1)
        scale_chunk(1, G - 1)
        dwait(sems0)
        sc_issue(G - 1, 1, sems1)
        dwait(sems1)

        @pl.when(gi < NG - 1)
        def _():
            load_idx_group(gi + 1)
            g_issue(0, 0, semg0)
        return carry

    load_idx_group(0)
    g_issue(0, 0, semg0)
    lax.fori_loop(0, NG, _group, None)
    plsc.subcore_barrier()

    pltpu.sync_copy(acc.at[pl.ds(s * RPS, RPS)],
                    out_hbm.at[c, pl.ds(s * RPS, RPS)])


_sc_agg = functools.partial(
    pl.kernel,
    out_type=jax.ShapeDtypeStruct((NC, APAD, D), jnp.float32),
    mesh=plsc.VectorSubcoreMesh(core_axis_name="c", subcore_axis_name="s"),
    scratch_types=[
        pltpu.VMEM((G, CH), jnp.int32),
        pltpu.VMEM((G, CH), jnp.int32),
        pltpu.VMEM((G, CH), jnp.float32),
        pltpu.VMEM((2, CH, D), jnp.float32),
        pltpu.VMEM((2, 80), jnp.int32),
        pltpu.VMEM_SHARED((APAD, D), jnp.float32),
        pltpu.SemaphoreType.DMA,
        pltpu.SemaphoreType.DMA,
        pltpu.SemaphoreType.DMA,
        pltpu.SemaphoreType.DMA,
    ],
)(_sc_body)


def kernel(feature, X, indices, values, weight, unique_idx, W_mask, W_ctx):
    pad = EP - E
    pad_idx = jnp.arange(pad, dtype=jnp.int32) % N
    rows2 = jnp.concatenate([indices[0], pad_idx]).reshape(EP // CH, CH)
    cols2 = jnp.concatenate([indices[1], pad_idx]).reshape(EP // CH, CH)
    vals2 = jnp.concatenate(
        [values, jnp.zeros((pad,), jnp.float32)]).reshape(EP // CH, CH)
    xp = X[N:]

    prompt = pl.pallas_call(
        _tc_prompt_body,
        out_shape=jax.ShapeDtypeStruct((M, D), jnp.float32),
    )(weight, xp)
    p_pad = jnp.pad(prompt, ((0, MPAD - M), (0, 0)))
    uidx_pad = jnp.concatenate(
        [unique_idx.astype(jnp.int32), jnp.arange(MPAD - M, dtype=jnp.int32)])

    agg2 = _sc_agg(rows2, cols2, vals2, X, p_pad, uidx_pad)

    pred_context, pred_mask = pl.pallas_call(
        _tc_out_body,
        out_shape=(
            jax.ShapeDtypeStruct((NPAD, D), jnp.float32),
            jax.ShapeDtypeStruct((N, D), jnp.float32),
        ),
    )(agg2, W_ctx, feature, W_mask)

    return (pred_mask, pred_context, pred_mask[-L:], pred_context[-L:], weight)


# speedup vs baseline: 12.2230x; 1.0148x over previous
import functools

import jax
import jax.numpy as jnp
from jax import lax
from jax.experimental import pallas as pl
from jax.experimental.pallas import tpu as pltpu
from jax.experimental.pallas import tpu_sc as plsc

N = 10000
L = 16
E = 320000
D = 128
M = 5000

NPAD = N + L
NC = 2
NS = 16
NW = NC * NS
CH = 128
EP = 327680
EW = EP // NW
NCHUNK = EW // CH
G = 16
NG = NCHUNK // G
MPAD = 5120
MW = MPAD // NW
APAD = 10112
RPS = APAD // NS


def _tc_prompt_body(w_ref, xp_ref, p_ref):
    w = w_ref[...]
    mx = jnp.max(w, axis=-1, keepdims=True)
    ex = jnp.exp(w - mx)
    sm = ex / jnp.sum(ex, axis=-1, keepdims=True)
    p_ref[...] = jnp.dot(sm, xp_ref[...], preferred_element_type=jnp.float32)


def _tc_out_body(agg2_ref, wctx_ref, feat_ref, wmask_ref,
                 ctx_ref, mask_ref, proc_ref, prom_ref):
    agg = agg2_ref[0, :NPAD] + agg2_ref[1, :NPAD]
    ctx = jnp.dot(agg, wctx_ref[...], preferred_element_type=jnp.float32)
    ctx_ref[...] = jnp.maximum(ctx, 0.0)
    msk = jnp.dot(feat_ref[...], wmask_ref[...], preferred_element_type=jnp.float32)
    mask_ref[...] = jnp.maximum(msk, 0.0)
    proc_ref[...] = ctx_ref[NPAD - L:]
    prom_ref[...] = mask_ref[N - L:]


def _sc_body(rows_hbm, cols_hbm, vals_hbm, x_hbm, p_hbm, uidx_hbm, out_hbm,
             colsg, rowsg, valsg, gbuf, puidx2, acc,
             semg0, semg1, sems0, sems1):
    c = lax.axis_index("c")
    s = lax.axis_index("s")
    w = s * NC + c

    def g_issue(k, b, sem):
        pltpu.async_copy(x_hbm.at[colsg.at[k]], gbuf.at[b], sem)

    def sc_issue(k, b, sem):
        pltpu.async_copy(gbuf.at[b], acc.at[rowsg.at[k]], sem, add=True)

    def dwait(sem):
        pltpu.make_async_copy(x_hbm.at[pl.ds(0, CH)], gbuf.at[0], sem).wait()

    def load_idx_group(gi):
        r0 = w * NCHUNK + gi * G
        pltpu.sync_copy(cols_hbm.at[pl.ds(r0, G)], colsg)
        pltpu.sync_copy(rows_hbm.at[pl.ds(r0, G)], rowsg)
        pltpu.sync_copy(vals_hbm.at[pl.ds(r0, G)], valsg)

    def scale_chunk(b, k):
        def _sg(g, carry):
            v16 = valsg[k, pl.ds(g * 16, 16)]
            for e16 in range(16):
                vb = jnp.full((16,), v16[e16], jnp.float32)
                e = g * 16 + e16
                for j in range(D // 16):
                    sl = (b, e, pl.ds(j * 16, 16))
                    gbuf[sl] = gbuf[sl] * vb
            return carry
        lax.fori_loop(0, CH // 16, _sg, None)

    def _zero_row(i, carry):
        for j in range(D // 16):
            gbuf[0, i, pl.ds(j * 16, 16)] = jnp.zeros((16,), jnp.float32)
        return carry
    lax.fori_loop(0, CH, _zero_row, None)
    for t in range(RPS // CH):
        pltpu.sync_copy(gbuf.at[0], acc.at[pl.ds(s * RPS + t * CH, CH)])
    rem = RPS % CH
    if rem:
        pltpu.sync_copy(gbuf.at[0, pl.ds(0, rem)],
                        acc.at[pl.ds(s * RPS + (RPS // CH) * CH, rem)])
    plsc.subcore_barrier()

    load_idx_group(0)
    g_issue(0, 0, semg0)

    for t in range(2):
        pltpu.sync_copy(p_hbm.at[pl.ds(w * MW + t * 80, 80)],
                        gbuf.at[1, pl.ds(0, 80)])
        pltpu.sync_copy(uidx_hbm.at[pl.ds(w * MW + t * 80, 80)], puidx2.at[t])
        pltpu.sync_copy(gbuf.at[1, pl.ds(0, 80)], acc.at[puidx2.at[t]], add=True)

    def _pair_mid(p, carry):
        a = 2 * p
        dwait(sems1)
        g_issue(a + 1, 1, semg1)
        dwait(semg0)
        scale_chunk(0, a)
        sc_issue(a, 0, sems0)
        dwait(semg1)
        scale_chunk(1, a + 1)
        dwait(sems0)
        g_issue(a + 2, 0, semg0)
        sc_issue(a + 1, 1, sems1)
        return carry

    def _group(gi, carry):
        g_issue(1, 1, semg1)
        dwait(semg0)
        scale_chunk(0, 0)
        sc_issue(0, 0, sems0)
        dwait(semg1)
        scale_chunk(1, 1)
        dwait(sems0)
        g_issue(2, 0, semg0)
        sc_issue(1, 1, sems1)

        lax.fori_loop(1, G // 2 - 1, _pair_mid, None)

        dwait(sems1)
        g_issue(G - 1, 1, semg1)
        dwait(semg0)
        scale_chunk(0, G - 2)
        sc_issue(G - 2, 0, sems0)
        dwait(semg1)
        scale_chunk(1, G - 1)
        dwait(sems0)
        sc_issue(G - 1, 1, sems1)
        dwait(sems1)

        @pl.when(gi < NG - 1)
        def _():
            load_idx_group(gi + 1)
            g_issue(0, 0, semg0)
        return carry

    lax.fori_loop(0, NG, _group, None)
    plsc.subcore_barrier()

    pltpu.sync_copy(acc.at[pl.ds(s * RPS, RPS)],
                    out_hbm.at[c, pl.ds(s * RPS, RPS)])


_sc_agg = functools.partial(
    pl.kernel,
    out_type=jax.ShapeDtypeStruct((NC, APAD, D), jnp.float32),
    mesh=plsc.VectorSubcoreMesh(core_axis_name="c", subcore_axis_name="s"),
    scratch_types=[
        pltpu.VMEM((G, CH), jnp.int32),
        pltpu.VMEM((G, CH), jnp.int32),
        pltpu.VMEM((G, CH), jnp.float32),
        pltpu.VMEM((2, CH, D), jnp.float32),
        pltpu.VMEM((2, 80), jnp.int32),
        pltpu.VMEM_SHARED((APAD, D), jnp.float32),
        pltpu.SemaphoreType.DMA,
        pltpu.SemaphoreType.DMA,
        pltpu.SemaphoreType.DMA,
        pltpu.SemaphoreType.DMA,
    ],
)(_sc_body)


def kernel(feature, X, indices, values, weight, unique_idx, W_mask, W_ctx):
    pad = EP - E
    pad_idx = jnp.arange(pad, dtype=jnp.int32) % N
    rows2 = jnp.concatenate([indices[0], pad_idx]).reshape(EP // CH, CH)
    cols2 = jnp.concatenate([indices[1], pad_idx]).reshape(EP // CH, CH)
    vals2 = jnp.concatenate(
        [values, jnp.zeros((pad,), jnp.float32)]).reshape(EP // CH, CH)
    xp = X[N:]

    prompt = pl.pallas_call(
        _tc_prompt_body,
        out_shape=jax.ShapeDtypeStruct((M, D), jnp.float32),
    )(weight, xp)
    p_pad = jnp.pad(prompt, ((0, MPAD - M), (0, 0)))
    uidx_pad = jnp.concatenate(
        [unique_idx.astype(jnp.int32), jnp.arange(MPAD - M, dtype=jnp.int32)])

    agg2 = _sc_agg(rows2, cols2, vals2, X, p_pad, uidx_pad)

    pred_context, pred_mask, pro_ctx, pro_mask = pl.pallas_call(
        _tc_out_body,
        out_shape=(
            jax.ShapeDtypeStruct((NPAD, D), jnp.float32),
            jax.ShapeDtypeStruct((N, D), jnp.float32),
            jax.ShapeDtypeStruct((L, D), jnp.float32),
            jax.ShapeDtypeStruct((L, D), jnp.float32),
        ),
    )(agg2, W_ctx, feature, W_mask)

    return (pred_mask, pred_context, pro_mask, pro_ctx, weight)


# speedup vs baseline: 12.6803x; 1.0374x over previous
import functools

import jax
import jax.numpy as jnp
from jax import lax
from jax.experimental import pallas as pl
from jax.experimental.pallas import tpu as pltpu
from jax.experimental.pallas import tpu_sc as plsc

N = 10000
L = 16
E = 320000
D = 128
M = 5000

NPAD = N + L
NC = 2
NS = 16
NW = NC * NS
CH = 128
EP = 327680
EW = EP // NW
NCHUNK = EW // CH
G = 40
NG = NCHUNK // G
MPAD = 5120
MW = MPAD // NW
APAD = 10112
RPS = APAD // NS


def _tc_prompt_body(w_ref, xp_ref, p_ref):
    w = w_ref[...]
    mx = jnp.max(w, axis=-1, keepdims=True)
    ex = jnp.exp(w - mx)
    sm = ex / jnp.sum(ex, axis=-1, keepdims=True)
    p_ref[...] = jnp.dot(sm, xp_ref[...], preferred_element_type=jnp.float32)


def _tc_out_body(agg2_ref, wctx_ref, feat_ref, wmask_ref,
                 ctx_ref, mask_ref, proc_ref, prom_ref):
    agg = agg2_ref[0, :NPAD] + agg2_ref[1, :NPAD]
    ctx = jnp.dot(agg, wctx_ref[...], preferred_element_type=jnp.float32)
    ctx_ref[...] = jnp.maximum(ctx, 0.0)
    msk = jnp.dot(feat_ref[...], wmask_ref[...], preferred_element_type=jnp.float32)
    mask_ref[...] = jnp.maximum(msk, 0.0)
    proc_ref[...] = ctx_ref[NPAD - L:]
    prom_ref[...] = mask_ref[N - L:]


def _sc_body(rows_hbm, cols_hbm, vals_hbm, x_hbm, p_hbm, uidx_hbm, out_hbm,
             colsg, rowsg, valsg, gbuf, puidx2, acc,
             semg0, semg1, sems0, sems1):
    c = lax.axis_index("c")
    s = lax.axis_index("s")
    w = s * NC + c

    def g_issue(k, b, sem):
        pltpu.async_copy(x_hbm.at[colsg.at[k]], gbuf.at[b], sem)

    def sc_issue(k, b, sem):
        pltpu.async_copy(gbuf.at[b], acc.at[rowsg.at[k]], sem, add=True)

    def dwait(sem):
        pltpu.make_async_copy(x_hbm.at[pl.ds(0, CH)], gbuf.at[0], sem).wait()

    def load_idx_group(gi):
        r0 = w * NCHUNK + gi * G
        pltpu.sync_copy(cols_hbm.at[pl.ds(r0, G)], colsg)
        pltpu.sync_copy(rows_hbm.at[pl.ds(r0, G)], rowsg)
        pltpu.sync_copy(vals_hbm.at[pl.ds(r0, G)], valsg)

    def scale_chunk(b, k, unroll=1):
        def _sg(g, carry):
            v16 = valsg[k, pl.ds(g * 16, 16)]
            for e16 in range(16):
                vb = jnp.full((16,), v16[e16], jnp.float32)
                e = g * 16 + e16
                for j in range(D // 16):
                    sl = (b, e, pl.ds(j * 16, 16))
                    gbuf[sl] = gbuf[sl] * vb
            return carry
        lax.fori_loop(0, CH // 16, _sg, None, unroll=unroll)

    def _zero_row(i, carry):
        for j in range(D // 16):
            gbuf[0, i, pl.ds(j * 16, 16)] = jnp.zeros((16,), jnp.float32)
        return carry
    lax.fori_loop(0, CH, _zero_row, None)
    zd = [pltpu.async_copy(gbuf.at[0], acc.at[pl.ds(s * RPS + t * CH, CH)],
                           semg0)
          for t in range(RPS // CH)]
    rem = RPS % CH
    if rem:
        zd.append(pltpu.async_copy(
            gbuf.at[0, pl.ds(0, rem)],
            acc.at[pl.ds(s * RPS + (RPS // CH) * CH, rem)], semg0))
    for d in zd:
        d.wait()
    plsc.subcore_barrier()

    load_idx_group(0)
    g_issue(0, 0, semg0)

    for t in range(2):
        pltpu.sync_copy(p_hbm.at[pl.ds(w * MW + t * 80, 80)],
                        gbuf.at[1, pl.ds(0, 80)])
        pltpu.sync_copy(uidx_hbm.at[pl.ds(w * MW + t * 80, 80)], puidx2.at[t])
        pltpu.sync_copy(gbuf.at[1, pl.ds(0, 80)], acc.at[puidx2.at[t]], add=True)

    def _pair_mid(p, carry):
        a = 2 * p
        dwait(sems1)
        g_issue(a + 1, 1, semg1)
        dwait(semg0)
        scale_chunk(0, a, unroll=2)
        sc_issue(a, 0, sems0)
        dwait(semg1)
        scale_chunk(1, a + 1, unroll=2)
        dwait(sems0)
        g_issue(a + 2, 0, semg0)
        sc_issue(a + 1, 1, sems1)
        return carry

    def _group(gi, carry):
        g_issue(1, 1, semg1)
        dwait(semg0)
        scale_chunk(0, 0)
        sc_issue(0, 0, sems0)
        dwait(semg1)
        scale_chunk(1, 1)
        dwait(sems0)
        g_issue(2, 0, semg0)
        sc_issue(1, 1, sems1)

        lax.fori_loop(1, G // 2 - 1, _pair_mid, None)

        dwait(sems1)
        g_issue(G - 1, 1, semg1)
        dwait(semg0)
        scale_chunk(0, G - 2)
        sc_issue(G - 2, 0, sems0)
        dwait(semg1)
        scale_chunk(1, G - 1)
        dwait(sems0)
        sc_issue(G - 1, 1, sems1)
        dwait(sems1)

        @pl.when(gi < NG - 1)
        def _():
            load_idx_group(gi + 1)
            g_issue(0, 0, semg0)
        return carry

    lax.fori_loop(0, NG, _group, None)
    plsc.subcore_barrier()

    pltpu.sync_copy(acc.at[pl.ds(s * RPS, RPS)],
                    out_hbm.at[c, pl.ds(s * RPS, RPS)])


_sc_agg = functools.partial(
    pl.kernel,
    out_type=jax.ShapeDtypeStruct((NC, APAD, D), jnp.float32),
    mesh=plsc.VectorSubcoreMesh(core_axis_name="c", subcore_axis_name="s"),
    scratch_types=[
        pltpu.VMEM((G, CH), jnp.int32),
        pltpu.VMEM((G, CH), jnp.int32),
        pltpu.VMEM((G, CH), jnp.float32),
        pltpu.VMEM((2, CH, D), jnp.float32),
        pltpu.VMEM((2, 80), jnp.int32),
        pltpu.VMEM_SHARED((APAD, D), jnp.float32),
        pltpu.SemaphoreType.DMA,
        pltpu.SemaphoreType.DMA,
        pltpu.SemaphoreType.DMA,
        pltpu.SemaphoreType.DMA,
    ],
)(_sc_body)


def kernel(feature, X, indices, values, weight, unique_idx, W_mask, W_ctx):
    pad = EP - E
    pad_idx = jnp.arange(pad, dtype=jnp.int32) % N
    rows2 = jnp.concatenate([indices[0], pad_idx]).reshape(EP // CH, CH)
    cols2 = jnp.concatenate([indices[1], pad_idx]).reshape(EP // CH, CH)
    vals2 = jnp.concatenate(
        [values, jnp.zeros((pad,), jnp.float32)]).reshape(EP // CH, CH)
    xp = X[N:]

    prompt = pl.pallas_call(
        _tc_prompt_body,
        out_shape=jax.ShapeDtypeStruct((M, D), jnp.float32),
    )(weight, xp)
    p_pad = jnp.pad(prompt, ((0, MPAD - M), (0, 0)))
    uidx_pad = jnp.concatenate(
        [unique_idx.astype(jnp.int32), jnp.arange(MPAD - M, dtype=jnp.int32)])

    agg2 = _sc_agg(rows2, cols2, vals2, X, p_pad, uidx_pad)

    pred_context, pred_mask, pro_ctx, pro_mask = pl.pallas_call(
        _tc_out_body,
        out_shape=(
            jax.ShapeDtypeStruct((NPAD, D), jnp.float32),
            jax.ShapeDtypeStruct((N, D), jnp.float32),
            jax.ShapeDtypeStruct((L, D), jnp.float32),
            jax.ShapeDtypeStruct((L, D), jnp.float32),
        ),
    )(agg2, W_ctx, feature, W_mask)

    return (pred_mask, pred_context, pro_mask, pro_ctx, weight)
